# baseline (speedup 1.0000x reference)
import jax
import jax.numpy as jnp
from jax.experimental import pallas as pl

OUT = 128
MASKING = -1000000000.0
BM = 2048

COL_BACK = 4 * 246 + 3 - 896
COL_FWD = 4 * 250 + 3 - 896
COL_LEFT = 4 * 251 + 3 - 896
COL_RIGHT = 4 * 255 + 3 - 896


def _mask_kernel(x_ref, o_ref):
    back = x_ref[:, COL_BACK:COL_BACK + 1] != 0
    fwd = x_ref[:, COL_FWD:COL_FWD + 1] != 0
    left = x_ref[:, COL_LEFT:COL_LEFT + 1] != 0
    right = x_ref[:, COL_RIGHT:COL_RIGHT + 1] != 0
    col = jax.lax.broadcasted_iota(jnp.int32, (BM, OUT), 1)
    hit = ((col == 1) & fwd) | ((col == 2) & back) \
        | ((col == 3) & left) | ((col == 4) & right)
    o_ref[...] = jnp.where(hit, MASKING, 1.0)


def kernel(x):
    B = x.shape[0]
    xb = jax.lax.bitcast_convert_type(x, jnp.uint8).reshape(B, 1024)
    return pl.pallas_call(
        _mask_kernel,
        grid=(B // BM,),
        in_specs=[pl.BlockSpec((BM, 128), lambda i: (i, 7))],
        out_specs=pl.BlockSpec((BM, OUT), lambda i: (i, 0)),
        out_shape=jax.ShapeDtypeStruct((B, OUT), jnp.float32),
    )(xb)


# speedup vs baseline: 8.7138x; 8.7138x over previous
import functools

import jax
import jax.numpy as jnp
from jax import lax
from jax.experimental import pallas as pl
from jax.experimental.pallas import tpu as pltpu
from jax.experimental.pallas import tpu_sc as plsc

B = 16384
OUT = 128
MASKING = -1000000000.0
NW = 32
RPW = B // NW
CHUNK = 128
NCHUNK = RPW // CHUNK

ACTION_SRC = ((1, 250 - 128), (2, 246 - 128), (3, 251 - 128), (4, 255 - 128))

_mesh = plsc.VectorSubcoreMesh(core_axis_name="c", subcore_axis_name="s")


@functools.partial(
    pl.kernel,
    mesh=_mesh,
    compiler_params=pltpu.CompilerParams(needs_layout_passes=False),
    out_type=jax.ShapeDtypeStruct((B, OUT), jnp.float32),
    scratch_types=[
        pltpu.VMEM((CHUNK, 128), jnp.float32),
        pltpu.VMEM((CHUNK, 128), jnp.float32),
        pltpu.VMEM((RPW, OUT), jnp.float32),
        pltpu.SemaphoreType.DMA,
        pltpu.SemaphoreType.DMA,
    ],
)
def _sc_mask(x_hbm, out_hbm, xs0, xs1, buf, sem0, sem1):
    wid = lax.axis_index("s") * 2 + lax.axis_index("c")
    base = wid * RPW
    xs = (xs0, xs1)
    sems = (sem0, sem1)

    def in_copy(c):
        return pltpu.make_async_copy(
            x_hbm.at[pl.ds(base + c * CHUNK, CHUNK), pl.ds(128, 128)],
            xs[c % 2], sems[c % 2])

    in_copy(0).start()
    in_copy(1).start()

    lane = lax.iota(jnp.int32, 16)
    ones = jnp.full((16,), 1.0, jnp.float32)

    def fill(r, carry):
        for k in range(8):
            buf[r, 16 * k:16 * (k + 1)] = ones
        return carry

    lax.fori_loop(0, RPW, fill, 0)

    for c in range(NCHUNK):
        in_copy(c).wait()
        src = xs[c % 2]

        def group(g, carry, c=c, src=src):
            rows = g * 16 + lane
            brows = c * CHUNK + rows
            for a, cond_col in ACTION_SRC:
                vals = plsc.load_gather(
                    src, [rows, jnp.full((16,), cond_col, jnp.int32)])
                out16 = jnp.where(vals == 1.0, MASKING, 1.0)
                plsc.store_scatter(
                    buf, [brows, jnp.full((16,), a, jnp.int32)], out16)
            return carry

        lax.fori_loop(0, CHUNK // 16, group, 0)
        if c + 2 < NCHUNK:
            in_copy(c + 2).start()

    pltpu.sync_copy(buf, out_hbm.at[pl.ds(base, RPW)])


def kernel(x):
    return _sc_mask(x)
